# baseline (speedup 1.0000x reference)
import jax
import jax.numpy as jnp
from jax.experimental import pallas as pl
from jax.experimental.pallas import tpu as pltpu

N = 10000
R = 2
INDIM = 128
OUTDIM = 128

TM = 200


def _fw_kernel(x_ref, w_ref, o_ref):
    for r in range(R):
        o_ref[r] = jnp.dot(x_ref[...], w_ref[r],
                           preferred_element_type=jnp.float32)


def _mm_kernel(a_ref, b_ref, o_ref):
    o_ref[...] = jnp.dot(a_ref[...], b_ref[...],
                         preferred_element_type=jnp.float32)


@jax.jit
def kernel(X, A, W_F):
    fw = pl.pallas_call(
        _fw_kernel,
        out_shape=jax.ShapeDtypeStruct((R, N, OUTDIM), jnp.float32),
    )(X, W_F)
    fw = fw.reshape(R * N, OUTDIM)

    out = pl.pallas_call(
        _mm_kernel,
        grid=(N // TM,),
        in_specs=[
            pl.BlockSpec((TM, R * N), lambda m: (m, 0)),
            pl.BlockSpec((R * N, OUTDIM), lambda m: (0, 0)),
        ],
        out_specs=pl.BlockSpec((TM, OUTDIM), lambda m: (m, 0)),
        out_shape=jax.ShapeDtypeStruct((N, OUTDIM), jnp.float32),
        compiler_params=pltpu.CompilerParams(
            dimension_semantics=("arbitrary",),
        ),
    )(A, fw)
    return out


# speedup vs baseline: 1.0353x; 1.0353x over previous
import jax
import jax.numpy as jnp
from jax.experimental import pallas as pl
from jax.experimental.pallas import tpu as pltpu

N = 10000
R = 2
INDIM = 128
OUTDIM = 128

TM = 200


def _fused_kernel(x_ref, w_ref, a_ref, o_ref, fw_ref):
    @pl.when(pl.program_id(0) == 0)
    def _compute_fw():
        for r in range(R):
            fw_ref[r * N:(r + 1) * N, :] = jnp.dot(
                x_ref[...], w_ref[r], preferred_element_type=jnp.float32)

    o_ref[...] = jnp.dot(a_ref[...], fw_ref[...],
                         preferred_element_type=jnp.float32)


@jax.jit
def kernel(X, A, W_F):
    return pl.pallas_call(
        _fused_kernel,
        grid=(N // TM,),
        in_specs=[
            pl.BlockSpec((N, INDIM), lambda m: (0, 0)),
            pl.BlockSpec((R, INDIM, OUTDIM), lambda m: (0, 0, 0)),
            pl.BlockSpec((TM, R * N), lambda m: (m, 0)),
        ],
        out_specs=pl.BlockSpec((TM, OUTDIM), lambda m: (m, 0)),
        out_shape=jax.ShapeDtypeStruct((N, OUTDIM), jnp.float32),
        scratch_shapes=[pltpu.VMEM((R * N, OUTDIM), jnp.float32)],
        compiler_params=pltpu.CompilerParams(
            dimension_semantics=("arbitrary",),
        ),
    )(X, W_F, A)
